# Initial kernel scaffold; baseline (speedup 1.0000x reference)
import jax
import jax.numpy as jnp
from jax.experimental import pallas as pl


def kernel(x, edge_index, batch, params):
    raise NotImplementedError("write your pallas kernel here")


# speedup vs baseline: 2.6677x; 2.6677x over previous
import functools

import jax
import jax.numpy as jnp
from jax import lax
from jax.experimental import pallas as pl
from jax.experimental.pallas import tpu as pltpu
from jax.experimental.pallas import tpu_sc as plsc

N = 50000
E = 800000
H = 64
G = 128
NUM_LAYERS = 8

NSC = 2
NTILE = 16
NP = 26624
NPAD = NSC * NP
TRASH = NP
ACC_ROWS = NP + 128
ROWS_PER_TILE = NP // NTILE
WB_CHUNKS = ROWS_PER_TILE // 128

KB = 16
NB = 25
CH = KB * NB
EPT = CH * 128
EPAD = NTILE * EPT

BLK = 512
GRID = NPAD // BLK


def _make_agg():
    mesh = plsc.VectorSubcoreMesh(core_axis_name="c", subcore_axis_name="s")

    @functools.partial(
        pl.kernel,
        out_type=jax.ShapeDtypeStruct((NPAD, H), jnp.float32),
        mesh=mesh,
        scratch_types=[
            pltpu.VMEM((KB, 128), jnp.int32),
            pltpu.VMEM((KB, 128), jnp.int32),
            pltpu.VMEM((128, H), jnp.float32),
            pltpu.VMEM((128, H), jnp.float32),
            pltpu.VMEM_SHARED((ACC_ROWS, H), jnp.float32),
            pltpu.SemaphoreType.DMA,
            pltpu.SemaphoreType.DMA,
        ],
        compiler_params=pltpu.CompilerParams(use_tc_tiling_on_sc=False),
    )
    def agg(h_hbm, srcw_hbm, dstw_hbm, zrows_hbm, out_hbm,
            sidx, didx, rows0, rows1, acc, sem0, sem1):
        c = lax.axis_index("c")
        s = lax.axis_index("s")
        w = s * NSC + c
        for k in range(WB_CHUNKS):
            pltpu.sync_copy(zrows_hbm, acc.at[pl.ds(s * ROWS_PER_TILE + k * 128, 128)])
        plsc.subcore_barrier()

        rows = (rows0, rows1)
        sems = (sem0, sem1)

        def block(b, carry):
            pltpu.sync_copy(srcw_hbm.at[w, pl.ds(b * KB, KB)], sidx)
            pltpu.sync_copy(dstw_hbm.at[w, pl.ds(b * KB, KB)], didx)
            cps = [None, None]
            cps[0] = pltpu.async_copy(h_hbm.at[sidx.at[0]], rows[0], sems[0])
            for j in range(KB):
                if j + 1 < KB:
                    cps[(j + 1) % 2] = pltpu.async_copy(
                        h_hbm.at[sidx.at[j + 1]], rows[(j + 1) % 2], sems[(j + 1) % 2])
                cps[j % 2].wait()
                pltpu.sync_copy(rows[j % 2], acc.at[didx.at[j]], add=True)
            return carry

        lax.fori_loop(0, NB, block, 0)
        plsc.subcore_barrier()
        for k in range(WB_CHUNKS):
            lo = s * ROWS_PER_TILE + k * 128
            pltpu.sync_copy(acc.at[pl.ds(lo, 128)],
                            out_hbm.at[pl.ds(c * NP + lo, 128)])

    return agg


_agg64 = _make_agg()


def _lin(x16, W1p):
    def body(x_ref, w_ref, o_ref):
        o_ref[...] = jnp.dot(x_ref[...], w_ref[...], preferred_element_type=jnp.float32)

    return pl.pallas_call(
        body,
        grid=(GRID,),
        in_specs=[
            pl.BlockSpec((BLK, 16), lambda i: (i, 0)),
            pl.BlockSpec((16, H), lambda i: (0, 0)),
        ],
        out_specs=pl.BlockSpec((BLK, H), lambda i: (i, 0)),
        out_shape=jax.ShapeDtypeStruct((NPAD, H), jnp.float32),
    )(x16, W1p)


def _mlp(hin, aggin, W1, W2, aux):
    def body(h_ref, a_ref, w1_ref, w2_ref, aux_ref, o_ref):
        h = h_ref[...] + a_ref[...]
        auxv = aux_ref[...]
        b1 = auxv[0:1, :]
        b2 = auxv[1:2, :]
        s2 = auxv[2:3, :]
        t2 = auxv[3:4, :]
        z = jnp.maximum(jnp.dot(h, w1_ref[...], preferred_element_type=jnp.float32) + b1, 0.0)
        z = jnp.maximum(jnp.dot(z, w2_ref[...], preferred_element_type=jnp.float32) + b2, 0.0)
        o_ref[...] = z * s2 + t2

    return pl.pallas_call(
        body,
        grid=(GRID,),
        in_specs=[
            pl.BlockSpec((BLK, H), lambda i: (i, 0)),
            pl.BlockSpec((BLK, H), lambda i: (i, 0)),
            pl.BlockSpec((H, H), lambda i: (0, 0)),
            pl.BlockSpec((H, H), lambda i: (0, 0)),
            pl.BlockSpec((8, H), lambda i: (0, 0)),
        ],
        out_specs=pl.BlockSpec((BLK, H), lambda i: (i, 0)),
        out_shape=jax.ShapeDtypeStruct((NPAD, H), jnp.float32),
    )(hin, aggin, W1, W2, aux)


def _pool(h, batch3):
    def body(h_ref, b_ref, sum_ref, cnt_ref):
        i = pl.program_id(0)

        @pl.when(i == 0)
        def _():
            sum_ref[...] = jnp.zeros_like(sum_ref)
            cnt_ref[...] = jnp.zeros_like(cnt_ref)

        b = jnp.reshape(b_ref[...], (BLK,))
        gids = lax.broadcasted_iota(jnp.int32, (G, BLK), 0)
        ohT = (gids == b[None, :]).astype(jnp.float32)
        sum_ref[...] += jnp.dot(ohT, h_ref[...], preferred_element_type=jnp.float32)
        cnt_ref[...] += jnp.broadcast_to(jnp.sum(ohT, axis=1, keepdims=True), (G, 128))

    return pl.pallas_call(
        body,
        grid=(GRID,),
        in_specs=[
            pl.BlockSpec((BLK, H), lambda i: (i, 0)),
            pl.BlockSpec((1, 1, BLK), lambda i: (i, 0, 0)),
        ],
        out_specs=[
            pl.BlockSpec((G, H), lambda i: (0, 0)),
            pl.BlockSpec((G, 128), lambda i: (0, 0)),
        ],
        out_shape=[
            jax.ShapeDtypeStruct((G, H), jnp.float32),
            jax.ShapeDtypeStruct((G, 128), jnp.float32),
        ],
    )(h, batch3)


def _head(sums, cnt, l1W, aux1, l2Wp, aux2):
    def body(s_ref, c_ref, w1_ref, a1_ref, w2_ref, a2_ref, o_ref):
        pooled = s_ref[...] / jnp.maximum(c_ref[...][:, 0:1], 1.0)
        l1b = a1_ref[...][0:1, :]
        l2b = a2_ref[...][0:1, :]
        z = jnp.maximum(jnp.dot(pooled, w1_ref[...], preferred_element_type=jnp.float32) + l1b, 0.0)
        o = jnp.dot(z, w2_ref[...], preferred_element_type=jnp.float32) + l2b
        mask = lax.broadcasted_iota(jnp.int32, (G, 128), 1) < 3
        om = jnp.where(mask, o, -1e30)
        mx = jnp.max(om, axis=1, keepdims=True)
        e = jnp.where(mask, jnp.exp(o - mx), 0.0)
        lse = jnp.log(jnp.sum(e, axis=1, keepdims=True))
        o_ref[...] = (o - mx - lse)[:, 0:3]

    return pl.pallas_call(
        body,
        in_specs=[
            pl.BlockSpec((G, H), lambda: (0, 0)),
            pl.BlockSpec((G, 128), lambda: (0, 0)),
            pl.BlockSpec((H, H), lambda: (0, 0)),
            pl.BlockSpec((8, H), lambda: (0, 0)),
            pl.BlockSpec((H, 128), lambda: (0, 0)),
            pl.BlockSpec((8, 128), lambda: (0, 0)),
        ],
        out_specs=pl.BlockSpec((G, 3), lambda: (0, 0)),
        out_shape=jax.ShapeDtypeStruct((G, 3), jnp.float32),
    )(sums, cnt, l1W, aux1, l2Wp, aux2)


def _fold(W1, b1, g1, be1, m1, v1, W2, b2, g2, be2, m2, v2):
    s1 = g1 / jnp.sqrt(v1 + 1e-5)
    t1 = be1 - m1 * s1
    W2p = s1[:, None] * W2
    b2p = b2 + t1 @ W2
    s2 = g2 / jnp.sqrt(v2 + 1e-5)
    t2 = be2 - m2 * s2
    aux = jnp.zeros((8, H), jnp.float32)
    aux = aux.at[0].set(b1).at[1].set(b2p).at[2].set(s2).at[3].set(t2)
    return W1, W2p, aux


def kernel(x, edge_index, batch, params):
    src = edge_index[0].astype(jnp.int32)
    dst = edge_index[1].astype(jnp.int32)

    srcp = jnp.concatenate([src, jnp.zeros((EPAD - E,), jnp.int32)])
    dstp = jnp.concatenate([dst, jnp.full((EPAD - E,), -1, jnp.int32)])
    dl0 = jnp.where((dstp >= 0) & (dstp < NP), dstp, TRASH)
    d1 = dstp - NP
    dl1 = jnp.where((d1 >= 0) & (d1 < NP), d1, TRASH)
    src_t = srcp.reshape(NTILE, CH, 128)
    srcw = jnp.repeat(src_t, NSC, axis=0)
    dstw = jnp.stack([dl0.reshape(NTILE, CH, 128),
                      dl1.reshape(NTILE, CH, 128)], axis=1).reshape(NSC * NTILE, CH, 128)

    zrows = jnp.zeros((128, H), jnp.float32)

    p = params
    x16 = jnp.pad(x, ((0, NPAD - N), (0, 15)))
    W1p = jnp.pad(p['c1_W1'], ((0, 15), (0, 0)))
    W1a, W2a, auxa = _fold(W1p, p['c1_b1'], p['c1_bn1_g'], p['c1_bn1_b'],
                           p['c1_bn1_m'], p['c1_bn1_v'], p['c1_W2'], p['c1_b2'],
                           p['c1_bn2_g'], p['c1_bn2_b'], p['c1_bn2_m'], p['c1_bn2_v'])
    h0 = _lin(x16, W1a)

    L = NUM_LAYERS - 1
    s1 = p['bn1_g'] / jnp.sqrt(p['bn1_v'] + 1e-5)
    t1 = p['bn1_b'] - p['bn1_m'] * s1
    W2s = s1[:, :, None] * p['Ws2']
    b2s = p['bs2'] + jnp.einsum('lh,lhk->lk', t1, p['Ws2'])
    s2 = p['bn2_g'] / jnp.sqrt(p['bn2_v'] + 1e-5)
    t2 = p['bn2_b'] - p['bn2_m'] * s2
    auxs = jnp.zeros((L, 8, H), jnp.float32)
    auxs = auxs.at[:, 0].set(p['bs1']).at[:, 1].set(b2s).at[:, 2].set(s2).at[:, 3].set(t2)

    eye = jnp.eye(H, dtype=jnp.float32)
    W1_stack = jnp.concatenate([eye[None], p['Ws1']], axis=0)
    W2_stack = jnp.concatenate([W2a[None], W2s], axis=0)
    aux_stack = jnp.concatenate([auxa[None], auxs], axis=0)

    n_iters = jnp.where(src[0] >= -1, NUM_LAYERS, NUM_LAYERS - 1).astype(jnp.int32)

    def cond(st):
        return st[0] < n_iters

    def body(st):
        i, h = st
        W1l = lax.dynamic_index_in_dim(W1_stack, i, keepdims=False)
        W2l = lax.dynamic_index_in_dim(W2_stack, i, keepdims=False)
        auxl = lax.dynamic_index_in_dim(aux_stack, i, keepdims=False)
        agg = _agg64(h, srcw, dstw, zrows)
        return i + 1, _mlp(h, agg, W1l, W2l, auxl)

    _, h = lax.while_loop(cond, body, (jnp.int32(0), h0))

    batchp = jnp.concatenate([batch.astype(jnp.int32), jnp.full((NPAD - N,), G, jnp.int32)])
    sums, cnt = _pool(h, batchp.reshape(GRID, 1, BLK))

    aux1 = jnp.zeros((8, H), jnp.float32).at[0].set(p['lin1_b'])
    l2Wp = jnp.pad(p['lin2_W'], ((0, 0), (0, 125)))
    aux2 = jnp.zeros((8, 128), jnp.float32).at[0, 0:3].set(p['lin2_b'])
    return _head(sums, cnt, p['lin1_W'], aux1, l2Wp, aux2)


# speedup vs baseline: 5.0295x; 1.8853x over previous
import functools

import jax
import jax.numpy as jnp
from jax import lax
from jax.experimental import pallas as pl
from jax.experimental.pallas import tpu as pltpu
from jax.experimental.pallas import tpu_sc as plsc

N = 50000
E = 800000
H = 64
G = 128
NUM_LAYERS = 8

NSC = 2
NTILE = 16
HC = H // NSC
NPAD = 53248
TRASH = 50176
ROWS_PER_TILE = NPAD // NTILE
WB_CHUNKS = ROWS_PER_TILE // 128

KB = 16
NB = 25
CH = KB * NB
EPT = CH * 128
EPAD = NTILE * EPT

BLK = 512
GRID = NPAD // BLK


def _make_agg():
    mesh = plsc.VectorSubcoreMesh(core_axis_name="c", subcore_axis_name="s")

    @functools.partial(
        pl.kernel,
        out_type=jax.ShapeDtypeStruct((NSC, NPAD, HC), jnp.float32),
        mesh=mesh,
        scratch_types=[
            pltpu.VMEM((KB, 128), jnp.int32),
            pltpu.VMEM((KB, 128), jnp.int32),
            pltpu.VMEM((128, HC), jnp.float32),
            pltpu.VMEM((128, HC), jnp.float32),
            pltpu.VMEM_SHARED((NPAD, HC), jnp.float32),
            pltpu.SemaphoreType.DMA,
            pltpu.SemaphoreType.DMA,
        ],
        compiler_params=pltpu.CompilerParams(use_tc_tiling_on_sc=False),
    )
    def agg(h2_hbm, srcw_hbm, dstw_hbm, zrows_hbm, out_hbm,
            sidx, didx, rows0, rows1, acc, sem0, sem1):
        c = lax.axis_index("c")
        s = lax.axis_index("s")
        for k in range(WB_CHUNKS):
            pltpu.sync_copy(zrows_hbm, acc.at[pl.ds(s * ROWS_PER_TILE + k * 128, 128)])
        plsc.subcore_barrier()

        hplane = h2_hbm.at[c]
        rows = (rows0, rows1)
        sems = (sem0, sem1)

        def block(b, carry):
            pltpu.sync_copy(srcw_hbm.at[s, pl.ds(b * KB, KB)], sidx)
            pltpu.sync_copy(dstw_hbm.at[s, pl.ds(b * KB, KB)], didx)
            cps = [None, None]
            cps[0] = pltpu.async_copy(hplane.at[sidx.at[0]], rows[0], sems[0])
            for j in range(KB):
                if j + 1 < KB:
                    cps[(j + 1) % 2] = pltpu.async_copy(
                        hplane.at[sidx.at[j + 1]], rows[(j + 1) % 2], sems[(j + 1) % 2])
                cps[j % 2].wait()
                pltpu.sync_copy(rows[j % 2], acc.at[didx.at[j]], add=True)
            return carry

        lax.fori_loop(0, NB, block, 0)
        plsc.subcore_barrier()
        for k in range(WB_CHUNKS):
            lo = s * ROWS_PER_TILE + k * 128
            pltpu.sync_copy(acc.at[pl.ds(lo, 128)], out_hbm.at[c, pl.ds(lo, 128)])

    return agg


_agg32 = _make_agg()


def _lin(x16, W1p):
    def body(x_ref, w_ref, o_ref, o2_ref):
        z = jnp.dot(x_ref[...], w_ref[...], preferred_element_type=jnp.float32)
        o_ref[...] = z
        o2_ref[0] = z[:, 0:HC]
        o2_ref[1] = z[:, HC:H]

    return pl.pallas_call(
        body,
        grid=(GRID,),
        in_specs=[
            pl.BlockSpec((BLK, 16), lambda i: (i, 0)),
            pl.BlockSpec((16, H), lambda i: (0, 0)),
        ],
        out_specs=[
            pl.BlockSpec((BLK, H), lambda i: (i, 0)),
            pl.BlockSpec((NSC, BLK, HC), lambda i: (0, i, 0)),
        ],
        out_shape=[
            jax.ShapeDtypeStruct((NPAD, H), jnp.float32),
            jax.ShapeDtypeStruct((NSC, NPAD, HC), jnp.float32),
        ],
    )(x16, W1p)


def _mlp(hin, agg2, W1, W2, aux):
    def body(h_ref, a_ref, w1_ref, w2_ref, aux_ref, o_ref, o2_ref):
        auxv = aux_ref[...]
        b1 = auxv[0:1, :]
        b2 = auxv[1:2, :]
        s2 = auxv[2:3, :]
        t2 = auxv[3:4, :]
        z = jnp.dot(h_ref[...], w1_ref[...], preferred_element_type=jnp.float32)
        z += jnp.dot(a_ref[0], w1_ref[0:HC, :], preferred_element_type=jnp.float32)
        z += jnp.dot(a_ref[1], w1_ref[HC:H, :], preferred_element_type=jnp.float32)
        z = jnp.maximum(z + b1, 0.0)
        z = jnp.maximum(jnp.dot(z, w2_ref[...], preferred_element_type=jnp.float32) + b2, 0.0)
        z = z * s2 + t2
        o_ref[...] = z
        o2_ref[0] = z[:, 0:HC]
        o2_ref[1] = z[:, HC:H]

    return pl.pallas_call(
        body,
        grid=(GRID,),
        in_specs=[
            pl.BlockSpec((BLK, H), lambda i: (i, 0)),
            pl.BlockSpec((NSC, BLK, HC), lambda i: (0, i, 0)),
            pl.BlockSpec((H, H), lambda i: (0, 0)),
            pl.BlockSpec((H, H), lambda i: (0, 0)),
            pl.BlockSpec((8, H), lambda i: (0, 0)),
        ],
        out_specs=[
            pl.BlockSpec((BLK, H), lambda i: (i, 0)),
            pl.BlockSpec((NSC, BLK, HC), lambda i: (0, i, 0)),
        ],
        out_shape=[
            jax.ShapeDtypeStruct((NPAD, H), jnp.float32),
            jax.ShapeDtypeStruct((NSC, NPAD, HC), jnp.float32),
        ],
    )(hin, agg2, W1, W2, aux)


def _pool(h, batch3):
    def body(h_ref, b_ref, sum_ref, cnt_ref):
        i = pl.program_id(0)

        @pl.when(i == 0)
        def _():
            sum_ref[...] = jnp.zeros_like(sum_ref)
            cnt_ref[...] = jnp.zeros_like(cnt_ref)

        b = jnp.reshape(b_ref[...], (BLK,))
        gids = lax.broadcasted_iota(jnp.int32, (G, BLK), 0)
        ohT = (gids == b[None, :]).astype(jnp.float32)
        sum_ref[...] += jnp.dot(ohT, h_ref[...], preferred_element_type=jnp.float32)
        cnt_ref[...] += jnp.broadcast_to(jnp.sum(ohT, axis=1, keepdims=True), (G, 128))

    return pl.pallas_call(
        body,
        grid=(GRID,),
        in_specs=[
            pl.BlockSpec((BLK, H), lambda i: (i, 0)),
            pl.BlockSpec((1, 1, BLK), lambda i: (i, 0, 0)),
        ],
        out_specs=[
            pl.BlockSpec((G, H), lambda i: (0, 0)),
            pl.BlockSpec((G, 128), lambda i: (0, 0)),
        ],
        out_shape=[
            jax.ShapeDtypeStruct((G, H), jnp.float32),
            jax.ShapeDtypeStruct((G, 128), jnp.float32),
        ],
    )(h, batch3)


def _head(sums, cnt, l1W, aux1, l2Wp, aux2):
    def body(s_ref, c_ref, w1_ref, a1_ref, w2_ref, a2_ref, o_ref):
        pooled = s_ref[...] / jnp.maximum(c_ref[...][:, 0:1], 1.0)
        l1b = a1_ref[...][0:1, :]
        l2b = a2_ref[...][0:1, :]
        z = jnp.maximum(jnp.dot(pooled, w1_ref[...], preferred_element_type=jnp.float32) + l1b, 0.0)
        o = jnp.dot(z, w2_ref[...], preferred_element_type=jnp.float32) + l2b
        mask = lax.broadcasted_iota(jnp.int32, (G, 128), 1) < 3
        om = jnp.where(mask, o, -1e30)
        mx = jnp.max(om, axis=1, keepdims=True)
        e = jnp.where(mask, jnp.exp(o - mx), 0.0)
        lse = jnp.log(jnp.sum(e, axis=1, keepdims=True))
        o_ref[...] = (o - mx - lse)[:, 0:3]

    return pl.pallas_call(
        body,
        in_specs=[
            pl.BlockSpec((G, H), lambda: (0, 0)),
            pl.BlockSpec((G, 128), lambda: (0, 0)),
            pl.BlockSpec((H, H), lambda: (0, 0)),
            pl.BlockSpec((8, H), lambda: (0, 0)),
            pl.BlockSpec((H, 128), lambda: (0, 0)),
            pl.BlockSpec((8, 128), lambda: (0, 0)),
        ],
        out_specs=pl.BlockSpec((G, 3), lambda: (0, 0)),
        out_shape=jax.ShapeDtypeStruct((G, 3), jnp.float32),
    )(sums, cnt, l1W, aux1, l2Wp, aux2)


def _fold(W1, b1, g1, be1, m1, v1, W2, b2, g2, be2, m2, v2):
    s1 = g1 / jnp.sqrt(v1 + 1e-5)
    t1 = be1 - m1 * s1
    W2p = s1[:, None] * W2
    b2p = b2 + t1 @ W2
    s2 = g2 / jnp.sqrt(v2 + 1e-5)
    t2 = be2 - m2 * s2
    aux = jnp.zeros((8, H), jnp.float32)
    aux = aux.at[0].set(b1).at[1].set(b2p).at[2].set(s2).at[3].set(t2)
    return W1, W2p, aux


def kernel(x, edge_index, batch, params):
    src = edge_index[0].astype(jnp.int32)
    dst = edge_index[1].astype(jnp.int32)

    srcp = jnp.concatenate([src, jnp.zeros((EPAD - E,), jnp.int32)])
    dstp = jnp.concatenate([dst, jnp.full((EPAD - E,), -1, jnp.int32)])
    lane = lax.rem(jnp.arange(EPAD, dtype=jnp.int32), jnp.int32(128))
    dl = jnp.where(dstp >= 0, dstp, TRASH + lane)
    srcw = srcp.reshape(NTILE, CH, 128)
    dstw = dl.reshape(NTILE, CH, 128)

    zrows = jnp.zeros((128, HC), jnp.float32)

    p = params
    x16 = jnp.pad(x, ((0, NPAD - N), (0, 15)))
    W1p = jnp.pad(p['c1_W1'], ((0, 15), (0, 0)))
    W1a, W2a, auxa = _fold(W1p, p['c1_b1'], p['c1_bn1_g'], p['c1_bn1_b'],
                           p['c1_bn1_m'], p['c1_bn1_v'], p['c1_W2'], p['c1_b2'],
                           p['c1_bn2_g'], p['c1_bn2_b'], p['c1_bn2_m'], p['c1_bn2_v'])
    h0, h02 = _lin(x16, W1a)

    L = NUM_LAYERS - 1
    s1 = p['bn1_g'] / jnp.sqrt(p['bn1_v'] + 1e-5)
    t1 = p['bn1_b'] - p['bn1_m'] * s1
    W2s = s1[:, :, None] * p['Ws2']
    b2s = p['bs2'] + jnp.einsum('lh,lhk->lk', t1, p['Ws2'])
    s2 = p['bn2_g'] / jnp.sqrt(p['bn2_v'] + 1e-5)
    t2 = p['bn2_b'] - p['bn2_m'] * s2
    auxs = jnp.zeros((L, 8, H), jnp.float32)
    auxs = auxs.at[:, 0].set(p['bs1']).at[:, 1].set(b2s).at[:, 2].set(s2).at[:, 3].set(t2)

    eye = jnp.eye(H, dtype=jnp.float32)
    W1_stack = jnp.concatenate([eye[None], p['Ws1']], axis=0)
    W2_stack = jnp.concatenate([W2a[None], W2s], axis=0)
    aux_stack = jnp.concatenate([auxa[None], auxs], axis=0)

    n_iters = jnp.where(src[0] >= -1, NUM_LAYERS, NUM_LAYERS - 1).astype(jnp.int32)

    def cond(st):
        return st[0] < n_iters

    def body(st):
        i, h, h2 = st
        W1l = lax.dynamic_index_in_dim(W1_stack, i, keepdims=False)
        W2l = lax.dynamic_index_in_dim(W2_stack, i, keepdims=False)
        auxl = lax.dynamic_index_in_dim(aux_stack, i, keepdims=False)
        agg2 = _agg32(h2, srcw, dstw, zrows)
        hn, hn2 = _mlp(h, agg2, W1l, W2l, auxl)
        return i + 1, hn, hn2

    _, h, _ = lax.while_loop(cond, body, (jnp.int32(0), h0, h02))

    batchp = jnp.concatenate([batch.astype(jnp.int32), jnp.full((NPAD - N,), G, jnp.int32)])
    sums, cnt = _pool(h, batchp.reshape(GRID, 1, BLK))

    aux1 = jnp.zeros((8, H), jnp.float32).at[0].set(p['lin1_b'])
    l2Wp = jnp.pad(p['lin2_W'], ((0, 0), (0, 125)))
    aux2 = jnp.zeros((8, 128), jnp.float32).at[0, 0:3].set(p['lin2_b'])
    return _head(sums, cnt, p['lin1_W'], aux1, l2Wp, aux2)


# speedup vs baseline: 5.5702x; 1.1075x over previous
import functools

import jax
import jax.numpy as jnp
from jax import lax
from jax.experimental import pallas as pl
from jax.experimental.pallas import tpu as pltpu
from jax.experimental.pallas import tpu_sc as plsc

N = 50000
E = 800000
H = 64
G = 128
NUM_LAYERS = 8

NSC = 2
NTILE = 16
HC = H // NSC
NPAD = 53248
TRASH = 50176
ROWS_PER_TILE = NPAD // NTILE
WB_CHUNKS = ROWS_PER_TILE // 128

KB = 40
NB = 10
CH = KB * NB
EPT = CH * 128
EPAD = NTILE * EPT

BLK = 512
GRID = NPAD // BLK


def _make_agg():
    mesh = plsc.VectorSubcoreMesh(core_axis_name="c", subcore_axis_name="s")

    @functools.partial(
        pl.kernel,
        out_type=jax.ShapeDtypeStruct((NSC, NPAD, HC), jnp.float32),
        mesh=mesh,
        scratch_types=[
            pltpu.VMEM((KB, 128), jnp.int32),
            pltpu.VMEM((KB, 128), jnp.int32),
            pltpu.VMEM((128, HC), jnp.float32),
            pltpu.VMEM((128, HC), jnp.float32),
            pltpu.VMEM((128, HC), jnp.float32),
            pltpu.VMEM_SHARED((NPAD, HC), jnp.float32),
            pltpu.SemaphoreType.DMA,
            pltpu.SemaphoreType.DMA,
            pltpu.SemaphoreType.DMA,
            pltpu.SemaphoreType.DMA,
            pltpu.SemaphoreType.DMA,
        ],
        compiler_params=pltpu.CompilerParams(use_tc_tiling_on_sc=False),
    )
    def agg(h2_hbm, srcw_hbm, dstw_hbm, zrows_hbm, out_hbm,
            sidx, didx, rows0, rows1, rows2, acc, semi0, semi1, sem0, sem1, sem2):
        c = lax.axis_index("c")
        s = lax.axis_index("s")
        for k in range(WB_CHUNKS):
            pltpu.sync_copy(zrows_hbm, acc.at[pl.ds(s * ROWS_PER_TILE + k * 128, 128)])
        plsc.subcore_barrier()

        hplane = h2_hbm.at[c]
        rows = (rows0, rows1, rows2)
        sems = (sem0, sem1, sem2)

        def block(b, carry):
            icp0 = pltpu.async_copy(srcw_hbm.at[s, pl.ds(b * KB, KB)], sidx, semi0)
            icp1 = pltpu.async_copy(dstw_hbm.at[s, pl.ds(b * KB, KB)], didx, semi1)
            icp0.wait()
            cps = [None, None, None]
            cps[0] = pltpu.async_copy(hplane.at[sidx.at[0]], rows[0], sems[0])
            cps[1] = pltpu.async_copy(hplane.at[sidx.at[1]], rows[1], sems[1])
            icp1.wait()
            for j in range(KB):
                if j + 2 < KB:
                    cps[(j + 2) % 3] = pltpu.async_copy(
                        hplane.at[sidx.at[j + 2]], rows[(j + 2) % 3], sems[(j + 2) % 3])
                cps[j % 3].wait()
                pltpu.sync_copy(rows[j % 3], acc.at[didx.at[j]], add=True)
            return carry

        lax.fori_loop(0, NB, block, 0)
        plsc.subcore_barrier()
        for k in range(WB_CHUNKS):
            lo = s * ROWS_PER_TILE + k * 128
            pltpu.sync_copy(acc.at[pl.ds(lo, 128)], out_hbm.at[c, pl.ds(lo, 128)])

    return agg


_agg32 = _make_agg()


def _lin(x16, W1p):
    def body(x_ref, w_ref, o_ref, o2_ref):
        z = jnp.dot(x_ref[...], w_ref[...], preferred_element_type=jnp.float32)
        o_ref[...] = z
        o2_ref[0] = z[:, 0:HC]
        o2_ref[1] = z[:, HC:H]

    return pl.pallas_call(
        body,
        grid=(GRID,),
        in_specs=[
            pl.BlockSpec((BLK, 16), lambda i: (i, 0)),
            pl.BlockSpec((16, H), lambda i: (0, 0)),
        ],
        out_specs=[
            pl.BlockSpec((BLK, H), lambda i: (i, 0)),
            pl.BlockSpec((NSC, BLK, HC), lambda i: (0, i, 0)),
        ],
        out_shape=[
            jax.ShapeDtypeStruct((NPAD, H), jnp.float32),
            jax.ShapeDtypeStruct((NSC, NPAD, HC), jnp.float32),
        ],
    )(x16, W1p)


def _mlp(hin, agg2, W1, W2, aux):
    def body(h_ref, a_ref, w1_ref, w2_ref, aux_ref, o_ref, o2_ref):
        auxv = aux_ref[...]
        b1 = auxv[0:1, :]
        b2 = auxv[1:2, :]
        s2 = auxv[2:3, :]
        t2 = auxv[3:4, :]
        z = jnp.dot(h_ref[...], w1_ref[...], preferred_element_type=jnp.float32)
        z += jnp.dot(a_ref[0], w1_ref[0:HC, :], preferred_element_type=jnp.float32)
        z += jnp.dot(a_ref[1], w1_ref[HC:H, :], preferred_element_type=jnp.float32)
        z = jnp.maximum(z + b1, 0.0)
        z = jnp.maximum(jnp.dot(z, w2_ref[...], preferred_element_type=jnp.float32) + b2, 0.0)
        z = z * s2 + t2
        o_ref[...] = z
        o2_ref[0] = z[:, 0:HC]
        o2_ref[1] = z[:, HC:H]

    return pl.pallas_call(
        body,
        grid=(GRID,),
        in_specs=[
            pl.BlockSpec((BLK, H), lambda i: (i, 0)),
            pl.BlockSpec((NSC, BLK, HC), lambda i: (0, i, 0)),
            pl.BlockSpec((H, H), lambda i: (0, 0)),
            pl.BlockSpec((H, H), lambda i: (0, 0)),
            pl.BlockSpec((8, H), lambda i: (0, 0)),
        ],
        out_specs=[
            pl.BlockSpec((BLK, H), lambda i: (i, 0)),
            pl.BlockSpec((NSC, BLK, HC), lambda i: (0, i, 0)),
        ],
        out_shape=[
            jax.ShapeDtypeStruct((NPAD, H), jnp.float32),
            jax.ShapeDtypeStruct((NSC, NPAD, HC), jnp.float32),
        ],
    )(hin, agg2, W1, W2, aux)


def _pool(h, batch3):
    def body(h_ref, b_ref, sum_ref, cnt_ref):
        i = pl.program_id(0)

        @pl.when(i == 0)
        def _():
            sum_ref[...] = jnp.zeros_like(sum_ref)
            cnt_ref[...] = jnp.zeros_like(cnt_ref)

        b = jnp.reshape(b_ref[...], (BLK,))
        gids = lax.broadcasted_iota(jnp.int32, (G, BLK), 0)
        ohT = (gids == b[None, :]).astype(jnp.float32)
        sum_ref[...] += jnp.dot(ohT, h_ref[...], preferred_element_type=jnp.float32)
        cnt_ref[...] += jnp.broadcast_to(jnp.sum(ohT, axis=1, keepdims=True), (G, 128))

    return pl.pallas_call(
        body,
        grid=(GRID,),
        in_specs=[
            pl.BlockSpec((BLK, H), lambda i: (i, 0)),
            pl.BlockSpec((1, 1, BLK), lambda i: (i, 0, 0)),
        ],
        out_specs=[
            pl.BlockSpec((G, H), lambda i: (0, 0)),
            pl.BlockSpec((G, 128), lambda i: (0, 0)),
        ],
        out_shape=[
            jax.ShapeDtypeStruct((G, H), jnp.float32),
            jax.ShapeDtypeStruct((G, 128), jnp.float32),
        ],
    )(h, batch3)


def _head(sums, cnt, l1W, aux1, l2Wp, aux2):
    def body(s_ref, c_ref, w1_ref, a1_ref, w2_ref, a2_ref, o_ref):
        pooled = s_ref[...] / jnp.maximum(c_ref[...][:, 0:1], 1.0)
        l1b = a1_ref[...][0:1, :]
        l2b = a2_ref[...][0:1, :]
        z = jnp.maximum(jnp.dot(pooled, w1_ref[...], preferred_element_type=jnp.float32) + l1b, 0.0)
        o = jnp.dot(z, w2_ref[...], preferred_element_type=jnp.float32) + l2b
        mask = lax.broadcasted_iota(jnp.int32, (G, 128), 1) < 3
        om = jnp.where(mask, o, -1e30)
        mx = jnp.max(om, axis=1, keepdims=True)
        e = jnp.where(mask, jnp.exp(o - mx), 0.0)
        lse = jnp.log(jnp.sum(e, axis=1, keepdims=True))
        o_ref[...] = (o - mx - lse)[:, 0:3]

    return pl.pallas_call(
        body,
        in_specs=[
            pl.BlockSpec((G, H), lambda: (0, 0)),
            pl.BlockSpec((G, 128), lambda: (0, 0)),
            pl.BlockSpec((H, H), lambda: (0, 0)),
            pl.BlockSpec((8, H), lambda: (0, 0)),
            pl.BlockSpec((H, 128), lambda: (0, 0)),
            pl.BlockSpec((8, 128), lambda: (0, 0)),
        ],
        out_specs=pl.BlockSpec((G, 3), lambda: (0, 0)),
        out_shape=jax.ShapeDtypeStruct((G, 3), jnp.float32),
    )(sums, cnt, l1W, aux1, l2Wp, aux2)


def _fold(W1, b1, g1, be1, m1, v1, W2, b2, g2, be2, m2, v2):
    s1 = g1 / jnp.sqrt(v1 + 1e-5)
    t1 = be1 - m1 * s1
    W2p = s1[:, None] * W2
    b2p = b2 + t1 @ W2
    s2 = g2 / jnp.sqrt(v2 + 1e-5)
    t2 = be2 - m2 * s2
    aux = jnp.zeros((8, H), jnp.float32)
    aux = aux.at[0].set(b1).at[1].set(b2p).at[2].set(s2).at[3].set(t2)
    return W1, W2p, aux


def kernel(x, edge_index, batch, params):
    src = edge_index[0].astype(jnp.int32)
    dst = edge_index[1].astype(jnp.int32)

    srcp = jnp.concatenate([src, jnp.zeros((EPAD - E,), jnp.int32)])
    dstp = jnp.concatenate([dst, jnp.full((EPAD - E,), -1, jnp.int32)])
    lane = lax.rem(jnp.arange(EPAD, dtype=jnp.int32), jnp.int32(128))
    dl = jnp.where(dstp >= 0, dstp, TRASH + lane)
    srcw = srcp.reshape(NTILE, CH, 128)
    dstw = dl.reshape(NTILE, CH, 128)

    zrows = jnp.zeros((128, HC), jnp.float32)

    p = params
    x16 = jnp.pad(x, ((0, NPAD - N), (0, 15)))
    W1p = jnp.pad(p['c1_W1'], ((0, 15), (0, 0)))
    W1a, W2a, auxa = _fold(W1p, p['c1_b1'], p['c1_bn1_g'], p['c1_bn1_b'],
                           p['c1_bn1_m'], p['c1_bn1_v'], p['c1_W2'], p['c1_b2'],
                           p['c1_bn2_g'], p['c1_bn2_b'], p['c1_bn2_m'], p['c1_bn2_v'])
    h0, h02 = _lin(x16, W1a)

    L = NUM_LAYERS - 1
    s1 = p['bn1_g'] / jnp.sqrt(p['bn1_v'] + 1e-5)
    t1 = p['bn1_b'] - p['bn1_m'] * s1
    W2s = s1[:, :, None] * p['Ws2']
    b2s = p['bs2'] + jnp.einsum('lh,lhk->lk', t1, p['Ws2'])
    s2 = p['bn2_g'] / jnp.sqrt(p['bn2_v'] + 1e-5)
    t2 = p['bn2_b'] - p['bn2_m'] * s2
    auxs = jnp.zeros((L, 8, H), jnp.float32)
    auxs = auxs.at[:, 0].set(p['bs1']).at[:, 1].set(b2s).at[:, 2].set(s2).at[:, 3].set(t2)

    eye = jnp.eye(H, dtype=jnp.float32)
    W1_stack = jnp.concatenate([eye[None], p['Ws1']], axis=0)
    W2_stack = jnp.concatenate([W2a[None], W2s], axis=0)
    aux_stack = jnp.concatenate([auxa[None], auxs], axis=0)

    n_iters = jnp.where(src[0] >= -1, NUM_LAYERS, NUM_LAYERS - 1).astype(jnp.int32)

    def cond(st):
        return st[0] < n_iters

    def body(st):
        i, h, h2 = st
        W1l = lax.dynamic_index_in_dim(W1_stack, i, keepdims=False)
        W2l = lax.dynamic_index_in_dim(W2_stack, i, keepdims=False)
        auxl = lax.dynamic_index_in_dim(aux_stack, i, keepdims=False)
        agg2 = _agg32(h2, srcw, dstw, zrows)
        hn, hn2 = _mlp(h, agg2, W1l, W2l, auxl)
        return i + 1, hn, hn2

    _, h, _ = lax.while_loop(cond, body, (jnp.int32(0), h0, h02))

    batchp = jnp.concatenate([batch.astype(jnp.int32), jnp.full((NPAD - N,), G, jnp.int32)])
    sums, cnt = _pool(h, batchp.reshape(GRID, 1, BLK))

    aux1 = jnp.zeros((8, H), jnp.float32).at[0].set(p['lin1_b'])
    l2Wp = jnp.pad(p['lin2_W'], ((0, 0), (0, 125)))
    aux2 = jnp.zeros((8, 128), jnp.float32).at[0, 0:3].set(p['lin2_b'])
    return _head(sums, cnt, p['lin1_W'], aux1, l2Wp, aux2)
